# baseline (speedup 1.0000x reference)
import functools

import jax
import jax.numpy as jnp
from jax import lax
from jax.experimental import pallas as pl
from jax.experimental.pallas import tpu as pltpu
from jax.experimental.pallas import tpu_sc as plsc

B, D, S = 4, 2048, 4096
D_BLK = 128
S_BLK = 128
NW = 32
N_D_CHUNKS = D // D_BLK
N_S_TILES = S // S_BLK // 2
N_VEC = S_BLK // 16


def _sc_body(x_hbm, pos_hbm, scale_hbm, out_hbm, pos_v, x_v, scale_v):
    wid = lax.axis_index("s") * 2 + lax.axis_index("c")
    dchunk = lax.rem(wid, N_D_CHUNKS)
    sgroup = wid // N_D_CHUNKS
    d0 = dchunk * D_BLK
    pltpu.sync_copy(scale_hbm.at[pl.ds(d0, D_BLK)], scale_v)
    iota = lax.iota(jnp.int32, 16)

    def s_tile(t, carry):
        s0 = (sgroup * N_S_TILES + t) * S_BLK
        pltpu.sync_copy(pos_hbm.at[pl.ds(s0, S_BLK), pl.ds(d0, D_BLK)], pos_v)
        pltpu.sync_copy(x_hbm.at[:, pl.ds(d0, D_BLK), pl.ds(s0, S_BLK)], x_v)

        def d_row(d, carry2):
            dcol = jnp.full((16,), d, dtype=jnp.int32)
            scv = plsc.load_gather(scale_v, [dcol])
            prow = []
            for j in range(N_VEC):
                pv = plsc.load_gather(pos_v, [iota + (16 * j), dcol])
                prow.append(pv * scv)
            for b in range(B):
                for j in range(N_VEC):
                    sl = pl.ds(16 * j, 16)
                    x_v[b, d, sl] = x_v[b, d, sl] + prow[j]
            return carry2

        lax.fori_loop(0, D_BLK, d_row, 0)
        pltpu.sync_copy(x_v, out_hbm.at[:, pl.ds(d0, D_BLK), pl.ds(s0, S_BLK)])
        return carry

    lax.fori_loop(0, N_S_TILES, s_tile, 0)


def kernel(x, adj_inp, cheb_polynomials, L_tilde, pos_table, scale):
    x3 = x.reshape(B, D, S)
    scale1 = scale.reshape(D)
    mesh = plsc.VectorSubcoreMesh(core_axis_name="c", subcore_axis_name="s")
    run = pl.kernel(
        _sc_body,
        mesh=mesh,
        compiler_params=pltpu.CompilerParams(needs_layout_passes=False),
        out_type=jax.ShapeDtypeStruct((B, D, S), jnp.float32),
        scratch_types=[
            pltpu.VMEM((S_BLK, D_BLK), jnp.float32),
            pltpu.VMEM((B, D_BLK, S_BLK), jnp.float32),
            pltpu.VMEM((D_BLK,), jnp.float32),
        ],
    )
    out = run(x3, pos_table, scale1)
    return out.reshape(B, D, S, 1)
